# baseline (speedup 1.0000x reference)
import functools

import jax
import jax.numpy as jnp
from jax import lax
from jax.experimental import pallas as pl
from jax.experimental.pallas import tpu as pltpu
from jax.experimental.pallas import tpu_sc as plsc

_VOCAB = 100
_EMB = 400
_OUT = 1000
_BITS = 8
_WORDS = 128
_BATCH = 16384

_NC = 2
_NS = 16
_NW = _NC * _NS
_BPW = _BATCH // _NW
_CHUNK = 128
_NCHUNK = _BPW // _CHUNK


def _table_kernel(tab_ref, w_ref, b_ref, out_ref):
    acc = jnp.dot(tab_ref[...], w_ref[...], preferred_element_type=jnp.float32)
    pred = ((acc + b_ref[...]) > 0.0).astype(jnp.float32)
    rows = lax.broadcasted_iota(jnp.int32, (_OUT, _WORDS), 0)
    cols = lax.broadcasted_iota(jnp.int32, (_OUT, _WORDS), 1)
    pack = jnp.where(rows % _WORDS == cols,
                     jnp.left_shift(1, rows // _WORDS), 0).astype(jnp.float32)
    packed = jnp.dot(pred, pack, preferred_element_type=jnp.float32)
    out_ref[...] = packed.astype(jnp.int32)


def _decision_table(table, W, b):
    return pl.pallas_call(
        _table_kernel,
        out_shape=jax.ShapeDtypeStruct((_VOCAB, _WORDS), jnp.int32),
    )(table, W, b.reshape(1, _OUT))


def _gather_rows(pt, idx3, nchunk):
    mesh = plsc.VectorSubcoreMesh(core_axis_name="c", subcore_axis_name="s")
    rows = _NW * nchunk * _CHUNK
    bpw = nchunk * _CHUNK

    @functools.partial(
        pl.kernel,
        mesh=mesh,
        out_type=jax.ShapeDtypeStruct((rows, _WORDS), jnp.int32),
        scratch_types=[
            pltpu.VMEM((nchunk, _CHUNK), jnp.int32),
            pltpu.VMEM((nchunk, _CHUNK, _WORDS), jnp.int32),
            pltpu.SemaphoreType.DMA,
            pltpu.SemaphoreType.DMA,
        ],
    )
    def k(pt_hbm, idx_hbm, out_hbm, idx_v, rows_v, gsem, ssem):
        wid = lax.axis_index("s") * _NC + lax.axis_index("c")
        pltpu.sync_copy(idx_hbm.at[wid], idx_v)
        gathers = [
            pltpu.async_copy(pt_hbm.at[idx_v.at[j]], rows_v.at[j], gsem)
            for j in range(nchunk)
        ]
        scatters = []
        for j in range(nchunk):
            gathers[j].wait()
            scatters.append(pltpu.async_copy(
                rows_v.at[j], out_hbm.at[pl.ds(wid * bpw + j * _CHUNK, _CHUNK)], ssem))
        for s in scatters:
            s.wait()

    return k(pt, idx3)


_UBLK = 2048


def _unpack_kernel(pk_ref, out_ref):
    blk = pk_ref[...]
    for k in range(_BITS):
        out_ref[:, pl.ds(_WORDS * k, _WORDS)] = ((blk >> k) & 1).astype(jnp.uint8)


def _unpack(packed):
    rows = packed.shape[0]
    return pl.pallas_call(
        _unpack_kernel,
        grid=(rows // _UBLK,),
        in_specs=[pl.BlockSpec((_UBLK, _WORDS), lambda i: (i, 0))],
        out_specs=pl.BlockSpec((_UBLK, _BITS * _WORDS), lambda i: (i, 0)),
        out_shape=jax.ShapeDtypeStruct((rows, _OUT), jnp.uint8),
    )(packed)


def kernel(inputs, embedding_var, W, b):
    pt = _decision_table(embedding_var, W, b)
    idx3 = inputs.astype(jnp.int32).reshape(_NW, _NCHUNK, _CHUNK)
    packed = _gather_rows(pt, idx3, _NCHUNK)
    bits = _unpack(packed)
    return bits.view(jnp.bool_)


# speedup vs baseline: 1.7387x; 1.7387x over previous
import functools

import jax
import jax.numpy as jnp
from jax import lax
from jax.experimental import pallas as pl
from jax.experimental.pallas import tpu as pltpu
from jax.experimental.pallas import tpu_sc as plsc

_VOCAB = 100
_EMB = 400
_OUT = 1000
_BITS = 8
_WORDS = 128
_BATCH = 16384

_NC = 2
_NS = 16
_NW = _NC * _NS
_BPW = _BATCH // _NW
_CHUNK = 128
_NCHUNK = _BPW // _CHUNK


def _table_kernel(tab_ref, w_ref, b_ref, out_ref):
    acc = jnp.dot(tab_ref[...], w_ref[...], preferred_element_type=jnp.float32)
    pred = ((acc + b_ref[...]) > 0.0).astype(jnp.float32)
    rows = lax.broadcasted_iota(jnp.int32, (_OUT, _WORDS), 0)
    cols = lax.broadcasted_iota(jnp.int32, (_OUT, _WORDS), 1)
    pack = jnp.where(rows % _WORDS == cols,
                     jnp.left_shift(1, rows // _WORDS), 0).astype(jnp.float32)
    packed = jnp.dot(pred, pack, preferred_element_type=jnp.float32)
    out_ref[...] = packed.astype(jnp.int32)


def _decision_table(table, W, b):
    return pl.pallas_call(
        _table_kernel,
        out_shape=jax.ShapeDtypeStruct((_VOCAB, _WORDS), jnp.int32),
    )(table, W, b.reshape(1, _OUT))


def _gather_rows(pt, idx3, nchunk):
    mesh = plsc.VectorSubcoreMesh(core_axis_name="c", subcore_axis_name="s")
    rows = _NW * nchunk * _CHUNK
    bpw = nchunk * _CHUNK

    @functools.partial(
        pl.kernel,
        mesh=mesh,
        out_type=jax.ShapeDtypeStruct((rows, _WORDS), jnp.int32),
        scratch_types=[
            pltpu.VMEM_SHARED((_VOCAB, _WORDS), jnp.int32),
            pltpu.VMEM((nchunk, _CHUNK), jnp.int32),
            pltpu.VMEM((nchunk, _CHUNK, _WORDS), jnp.int32),
            pltpu.SemaphoreType.DMA,
            pltpu.SemaphoreType.DMA,
        ],
    )
    def k(pt_hbm, idx_hbm, out_hbm, pt_s, idx_v, rows_v, gsem, ssem):
        sid = lax.axis_index("s")
        wid = sid * _NC + lax.axis_index("c")
        @pl.when(sid == 0)
        def _():
            pltpu.sync_copy(pt_hbm, pt_s)
        pltpu.sync_copy(idx_hbm.at[wid], idx_v)
        plsc.subcore_barrier()
        gathers = [
            pltpu.async_copy(pt_s.at[idx_v.at[j]], rows_v.at[j], gsem)
            for j in range(nchunk)
        ]
        scatters = []
        for j in range(nchunk):
            gathers[j].wait()
            scatters.append(pltpu.async_copy(
                rows_v.at[j], out_hbm.at[pl.ds(wid * bpw + j * _CHUNK, _CHUNK)], ssem))
        for s in scatters:
            s.wait()

    return k(pt, idx3)


_UBLK = 2048


def _unpack_kernel(pk_ref, out_ref):
    blk_t = pk_ref[...].T
    for k in range(_BITS):
        out_ref[pl.ds(_WORDS * k, _WORDS), :] = ((blk_t >> k) & 1).astype(jnp.uint8)


def _unpack_t(packed):
    rows = packed.shape[0]
    return pl.pallas_call(
        _unpack_kernel,
        grid=(rows // _UBLK,),
        in_specs=[pl.BlockSpec((_UBLK, _WORDS), lambda i: (i, 0))],
        out_specs=pl.BlockSpec((_BITS * _WORDS, _UBLK), lambda i: (0, i)),
        out_shape=jax.ShapeDtypeStruct((_BITS * _WORDS, rows), jnp.uint8),
    )(packed)


def kernel(inputs, embedding_var, W, b):
    pt = _decision_table(embedding_var, W, b)
    idx3 = inputs.astype(jnp.int32).reshape(_NW, _NCHUNK, _CHUNK)
    packed = _gather_rows(pt, idx3, _NCHUNK)
    bits_t = _unpack_t(packed)
    return (bits_t[:_OUT] != 0).T


# speedup vs baseline: 1.8048x; 1.0380x over previous
import functools

import jax
import jax.numpy as jnp
from jax import lax
from jax.experimental import pallas as pl
from jax.experimental.pallas import tpu as pltpu
from jax.experimental.pallas import tpu_sc as plsc

_VOCAB = 100
_EMB = 400
_OUT = 1000
_BITS = 8
_WORDS = 128
_BATCH = 16384

_NC = 2
_NS = 16
_NW = _NC * _NS
_BPW = _BATCH // _NW
_CHUNK = 128
_NCHUNK = _BPW // _CHUNK


def _table_kernel(tab_ref, w_ref, b_ref, out_ref):
    acc = jnp.dot(tab_ref[...], w_ref[...], preferred_element_type=jnp.float32)
    pred = ((acc + b_ref[...]) > 0.0).astype(jnp.float32)
    rows = lax.broadcasted_iota(jnp.int32, (_OUT, _WORDS), 0)
    cols = lax.broadcasted_iota(jnp.int32, (_OUT, _WORDS), 1)
    pack = jnp.where(rows % _WORDS == cols,
                     jnp.left_shift(1, rows // _WORDS), 0).astype(jnp.float32)
    packed = jnp.dot(pred, pack, preferred_element_type=jnp.float32)
    out_ref[...] = packed.astype(jnp.int32)


def _decision_table(table, W, b):
    return pl.pallas_call(
        _table_kernel,
        out_shape=jax.ShapeDtypeStruct((_VOCAB, _WORDS), jnp.int32),
    )(table, W, b.reshape(1, _OUT))


def _gather_rows(pt, idx3, nchunk):
    mesh = plsc.VectorSubcoreMesh(core_axis_name="c", subcore_axis_name="s")
    rows = _NW * nchunk * _CHUNK
    bpw = nchunk * _CHUNK

    @functools.partial(
        pl.kernel,
        mesh=mesh,
        out_type=jax.ShapeDtypeStruct((rows, _WORDS), jnp.int32),
        scratch_types=[
            pltpu.VMEM_SHARED((_VOCAB, _WORDS), jnp.int32),
            pltpu.VMEM((nchunk, _CHUNK), jnp.int32),
            pltpu.VMEM((nchunk, _CHUNK, _WORDS), jnp.int32),
            pltpu.SemaphoreType.DMA,
            pltpu.SemaphoreType.DMA,
        ],
    )
    def k(pt_hbm, idx_hbm, out_hbm, pt_s, idx_v, rows_v, gsem, ssem):
        sid = lax.axis_index("s")
        wid = sid * _NC + lax.axis_index("c")
        @pl.when(sid == 0)
        def _():
            pltpu.sync_copy(pt_hbm, pt_s)
        pltpu.sync_copy(idx_hbm.at[wid], idx_v)
        plsc.subcore_barrier()
        gathers = [
            pltpu.async_copy(pt_s.at[idx_v.at[j]], rows_v.at[j], gsem)
            for j in range(nchunk)
        ]
        scatters = []
        for j in range(nchunk):
            gathers[j].wait()
            scatters.append(pltpu.async_copy(
                rows_v.at[j], out_hbm.at[pl.ds(wid * bpw + j * _CHUNK, _CHUNK)], ssem))
        for s in scatters:
            s.wait()

    return k(pt, idx3)


_UBLK = 4096


def _unpack_kernel(pk_ref, out_ref):
    blk_t = pk_ref[...].T
    for k in range(_BITS):
        out_ref[pl.ds(_WORDS * k, _WORDS), :] = ((blk_t >> k) & 1).astype(jnp.uint8)


def _unpack_t(packed):
    rows = packed.shape[0]
    return pl.pallas_call(
        _unpack_kernel,
        grid=(rows // _UBLK,),
        in_specs=[pl.BlockSpec((_UBLK, _WORDS), lambda i: (i, 0))],
        out_specs=pl.BlockSpec((_BITS * _WORDS, _UBLK), lambda i: (0, i)),
        out_shape=jax.ShapeDtypeStruct((_BITS * _WORDS, rows), jnp.uint8),
    )(packed)


def kernel(inputs, embedding_var, W, b):
    pt = _decision_table(embedding_var, W, b)
    idx3 = inputs.astype(jnp.int32).reshape(_NW, _NCHUNK, _CHUNK)
    packed = _gather_rows(pt, idx3, _NCHUNK)
    bits_t = _unpack_t(packed)
    return (bits_t[:_OUT] != 0).T


# speedup vs baseline: 1.8096x; 1.0027x over previous
import functools

import jax
import jax.numpy as jnp
from jax import lax
from jax.experimental import pallas as pl
from jax.experimental.pallas import tpu as pltpu
from jax.experimental.pallas import tpu_sc as plsc

_VOCAB = 100
_EMB = 400
_OUT = 1000
_BITS = 8
_WORDS = 128
_BATCH = 16384

_NC = 2
_NS = 16
_NW = _NC * _NS
_BPW = _BATCH // _NW
_CHUNK = 128
_NCHUNK = _BPW // _CHUNK


def _table_kernel(tab_ref, w_ref, b_ref, out_ref):
    acc = jnp.dot(tab_ref[...], w_ref[...], preferred_element_type=jnp.float32)
    pred = ((acc + b_ref[...]) > 0.0).astype(jnp.float32)
    rows = lax.broadcasted_iota(jnp.int32, (_OUT, _WORDS), 0)
    cols = lax.broadcasted_iota(jnp.int32, (_OUT, _WORDS), 1)
    pack = jnp.where(rows % _WORDS == cols,
                     jnp.left_shift(1, rows // _WORDS), 0).astype(jnp.float32)
    packed = jnp.dot(pred, pack, preferred_element_type=jnp.float32)
    out_ref[...] = packed.astype(jnp.int32)


def _decision_table(table, W, b):
    return pl.pallas_call(
        _table_kernel,
        out_shape=jax.ShapeDtypeStruct((_VOCAB, _WORDS), jnp.int32),
    )(table, W, b.reshape(1, _OUT))


def _gather_rows(pt, idx3, nchunk):
    mesh = plsc.VectorSubcoreMesh(core_axis_name="c", subcore_axis_name="s")
    rows = _NW * nchunk * _CHUNK
    bpw = nchunk * _CHUNK

    @functools.partial(
        pl.kernel,
        mesh=mesh,
        out_type=jax.ShapeDtypeStruct((rows, _WORDS), jnp.int32),
        scratch_types=[
            pltpu.VMEM_SHARED((_VOCAB, _WORDS), jnp.int32),
            pltpu.VMEM((nchunk, _CHUNK), jnp.int32),
            pltpu.VMEM((nchunk, _CHUNK, _WORDS), jnp.int32),
            pltpu.SemaphoreType.DMA,
            pltpu.SemaphoreType.DMA,
        ],
    )
    def k(pt_hbm, idx_hbm, out_hbm, pt_s, idx_v, rows_v, gsem, ssem):
        sid = lax.axis_index("s")
        wid = sid * _NC + lax.axis_index("c")
        @pl.when(sid == 0)
        def _():
            pltpu.sync_copy(pt_hbm, pt_s)
        pltpu.sync_copy(idx_hbm.at[wid], idx_v)
        plsc.subcore_barrier()
        gathers = [
            pltpu.async_copy(pt_s.at[idx_v.at[j]], rows_v.at[j], gsem)
            for j in range(nchunk)
        ]
        scatters = []
        for j in range(nchunk):
            gathers[j].wait()
            scatters.append(pltpu.async_copy(
                rows_v.at[j], out_hbm.at[pl.ds(wid * bpw + j * _CHUNK, _CHUNK)], ssem))
        for s in scatters:
            s.wait()

    return k(pt, idx3)


_UBLK = 8192


def _unpack_kernel(pk_ref, out_ref):
    blk_t = pk_ref[...].T
    for k in range(_BITS):
        out_ref[pl.ds(_WORDS * k, _WORDS), :] = ((blk_t >> k) & 1).astype(jnp.uint8)


def _unpack_t(packed):
    rows = packed.shape[0]
    return pl.pallas_call(
        _unpack_kernel,
        grid=(rows // _UBLK,),
        in_specs=[pl.BlockSpec((_UBLK, _WORDS), lambda i: (i, 0))],
        out_specs=pl.BlockSpec((_BITS * _WORDS, _UBLK), lambda i: (0, i)),
        out_shape=jax.ShapeDtypeStruct((_BITS * _WORDS, rows), jnp.uint8),
    )(packed)


def kernel(inputs, embedding_var, W, b):
    pt = _decision_table(embedding_var, W, b)
    idx3 = inputs.astype(jnp.int32).reshape(_NW, _NCHUNK, _CHUNK)
    packed = _gather_rows(pt, idx3, _NCHUNK)
    bits_t = _unpack_t(packed)
    return (bits_t[:_OUT] != 0).T
